# baseline (speedup 1.0000x reference)
import functools

import jax
import jax.numpy as jnp
from jax import lax
from jax.experimental import pallas as pl
from jax.experimental.pallas import tpu as pltpu
from jax.experimental.pallas import tpu_sc as plsc

B = 4096
H = 200
E = 64
OUTD = 128
HALF = H // 2


def _make_sc_pool():
    info = plsc.get_sparse_core_info()
    nc, ns = info.num_cores, info.num_subcores
    nw = nc * ns
    bpw = B // nw
    mesh = plsc.VectorSubcoreMesh(core_axis_name="c", subcore_axis_name="s")

    @functools.partial(
        pl.kernel,
        out_type=jax.ShapeDtypeStruct((B, E), jnp.float32),
        mesh=mesh,
        scratch_types=[
            pltpu.VMEM((bpw, 2, HALF), jnp.int32),
            pltpu.VMEM((2, 2, HALF, E), jnp.float32),
            pltpu.VMEM((bpw, E), jnp.float32),
            pltpu.SemaphoreType.DMA,
            pltpu.SemaphoreType.DMA,
        ],
        compiler_params=pltpu.CompilerParams(use_tc_tiling_on_sc=False),
    )
    def pool(x_hbm, table_hbm, out_hbm, idx_v, rows_v, pooled_v, sem0, sem1):
        sems = (sem0, sem1)
        wid = lax.axis_index("s") * nc + lax.axis_index("c")
        base = wid * bpw
        pltpu.sync_copy(x_hbm.at[pl.ds(base, bpw)], idx_v)

        def start(b, par):
            for h in range(2):
                pltpu.async_copy(
                    table_hbm.at[idx_v.at[b, h]], rows_v.at[par, h], sems[par]
                )

        def wait(b, par):
            for h in range(2):
                pltpu.make_async_copy(
                    table_hbm.at[idx_v.at[b, h]], rows_v.at[par, h], sems[par]
                ).wait()

        start(0, 0)

        def outer(g, _):
            for par in range(2):
                b = 2 * g + par

                @pl.when(b + 1 < bpw)
                def _():
                    start(b + 1, (par + 1) % 2)

                wait(b, par)

                def inner(r, accs):
                    new = []
                    for h in range(2):
                        for j in range(E // 16):
                            new.append(
                                accs[h * (E // 16) + j]
                                + rows_v[par, h, r, pl.ds(j * 16, 16)]
                            )
                    return tuple(new)

                zero = jnp.zeros((16,), jnp.float32)
                accs = lax.fori_loop(0, HALF, inner, (zero,) * (2 * (E // 16)))
                for j in range(E // 16):
                    pooled_v[b, pl.ds(j * 16, 16)] = accs[j] + accs[E // 16 + j]
            return 0

        lax.fori_loop(0, bpw // 2, outer, 0)
        pltpu.sync_copy(pooled_v, out_hbm.at[pl.ds(base, bpw)])

    return pool


def _tc_proj(pooled_sum, W, b):
    blk = 512

    def body(p_ref, w_ref, b_ref, o_ref):
        o_ref[...] = (
            jnp.dot(
                p_ref[...] * (1.0 / H), w_ref[...],
                preferred_element_type=jnp.float32,
            )
            + b_ref[...]
        )

    return pl.pallas_call(
        body,
        grid=(B // blk,),
        in_specs=[
            pl.BlockSpec((blk, E), lambda i: (i, 0)),
            pl.BlockSpec((E, OUTD), lambda i: (0, 0)),
            pl.BlockSpec((1, OUTD), lambda i: (0, 0)),
        ],
        out_specs=pl.BlockSpec((blk, OUTD), lambda i: (i, 0)),
        out_shape=jax.ShapeDtypeStruct((B, OUTD), jnp.float32),
    )(pooled_sum, W, b.reshape(1, OUTD))


def kernel(x, table, W, b):
    x3 = x.astype(jnp.int32).reshape(B, 2, HALF)
    pooled_sum = _make_sc_pool()(x3, table)
    return _tc_proj(pooled_sum, W, b)
